# baseline (speedup 1.0000x reference)
import functools

import jax
import jax.numpy as jnp
from jax import lax
from jax.experimental import pallas as pl
from jax.experimental.pallas import tpu as pltpu
from jax.experimental.pallas import tpu_sc as plsc

B = 128
D = 128
M = 100000
K = 4096

H = 51200
MC2 = 12800

NC = 2
NS = 16
NW = NC * NS
ROWS_PER_TILE = B // NW


def _sims_body(feat_ref, mem_lo_ref, mem_hi_ref, out_ref):
    feat = feat_ref[...].astype(jnp.bfloat16)

    def halfdot(mem_ref):
        acc = lax.dot_general(
            feat,
            mem_ref[...].astype(jnp.bfloat16),
            dimension_numbers=(((1,), (1,)), ((), ())),
            preferred_element_type=jnp.float32,
        )
        return lax.bitcast_convert_type(
            acc.astype(jnp.bfloat16).astype(jnp.float32), jnp.int32
        )

    lo = halfdot(mem_lo_ref)
    hi = halfdot(mem_hi_ref)
    out_ref[...] = lax.bitwise_or(lax.shift_right_logical(lo, 16), hi)


def _sims_packed(feature, memory):
    n_blocks = H // MC2
    return pl.pallas_call(
        _sims_body,
        grid=(n_blocks,),
        in_specs=[
            pl.BlockSpec((B, D), lambda i: (0, 0)),
            pl.BlockSpec((MC2, D), lambda i: (i, 0)),
            pl.BlockSpec((MC2, D), lambda i: (i + H // MC2, 0)),
        ],
        out_specs=pl.BlockSpec((B, MC2), lambda i: (0, i)),
        out_shape=jax.ShapeDtypeStruct((B, H), jnp.int32),
    )(feature, memory, memory)


_mesh = plsc.VectorSubcoreMesh(core_axis_name="c", subcore_axis_name="s")


@functools.partial(
    pl.kernel,
    out_type=jax.ShapeDtypeStruct((B, K), jnp.float32),
    mesh=_mesh,
    compiler_params=pltpu.CompilerParams(
        needs_layout_passes=False,
        disable_bounds_checks=True,
        disable_semaphore_checks=True,
    ),
    scratch_types=[
        pltpu.VMEM((H,), jnp.int32),
        pltpu.VMEM((H,), jnp.int32),
        pltpu.VMEM((ROWS_PER_TILE, K), jnp.int32),
        pltpu.VMEM((K,), jnp.float32),
        pltpu.SemaphoreType.DMA,
        pltpu.SemaphoreType.DMA,
    ],
)
def _gather(sims_hbm, idx_hbm, out_hbm, buf0, buf1, idx_v, out_v, sem0, sem1):
    wid = lax.axis_index("s") * NC + lax.axis_index("c")
    b0 = wid * ROWS_PER_TILE
    bufs = [buf0, buf1]
    sems = [sem0, sem1]

    pltpu.sync_copy(idx_hbm.at[pl.ds(b0, ROWS_PER_TILE)], idx_v)
    pending = pltpu.async_copy(sims_hbm.at[b0], buf0, sem0)
    for j in range(ROWS_PER_TILE):
        if j + 1 < ROWS_PER_TILE:
            nxt = pltpu.async_copy(
                sims_hbm.at[b0 + j + 1], bufs[(j + 1) % 2], sems[(j + 1) % 2]
            )
        pending.wait()
        buf = bufs[j % 2]

        @plsc.parallel_loop(0, K // 16, unroll=16)
        def _(t):
            iv = idx_v[j, pl.ds(t * 16, 16)]
            hi_half = lax.ge(iv, H)
            wi = jnp.where(hi_half, iv - H, iv)
            word = plsc.load_gather(buf, [wi])
            bits = jnp.where(hi_half, word, lax.shift_left(word, 16))
            f32bits = lax.bitwise_and(bits, jnp.int32(-65536))
            out_v[pl.ds(t * 16, 16)] = plsc.bitcast(f32bits, jnp.float32)
        pltpu.sync_copy(out_v, out_hbm.at[b0 + j])
        if j + 1 < ROWS_PER_TILE:
            pending = nxt


def kernel(feature, memory, selected_neg_idx):
    idx = selected_neg_idx.reshape(B, K).astype(jnp.int32)
    sims_words = _sims_packed(feature, memory)
    out = _gather(sims_words, idx)
    return out[..., None]


# speedup vs baseline: 1.0148x; 1.0148x over previous
import functools

import jax
import jax.numpy as jnp
from jax import lax
from jax.experimental import pallas as pl
from jax.experimental.pallas import tpu as pltpu
from jax.experimental.pallas import tpu_sc as plsc

B = 128
D = 128
M = 100000
K = 4096

H = 51200
MC2 = 12800

NC = 2
NS = 16
NW = NC * NS
ROWS_PER_TILE = B // NW


def _sims_body(feat_ref, mem_lo_ref, mem_hi_ref, out_ref):
    feat = feat_ref[...].astype(jnp.bfloat16)

    def halfdot(mem_ref):
        acc = lax.dot_general(
            feat,
            mem_ref[...].astype(jnp.bfloat16),
            dimension_numbers=(((1,), (1,)), ((), ())),
            preferred_element_type=jnp.float32,
        )
        return lax.bitcast_convert_type(
            acc.astype(jnp.bfloat16).astype(jnp.float32), jnp.int32
        )

    lo = halfdot(mem_lo_ref)
    hi = halfdot(mem_hi_ref)
    out_ref[...] = lax.bitwise_or(lax.shift_right_logical(lo, 16), hi)


def _sims_packed(feature, memory):
    n_blocks = H // MC2
    return pl.pallas_call(
        _sims_body,
        grid=(n_blocks,),
        in_specs=[
            pl.BlockSpec((B, D), lambda i: (0, 0)),
            pl.BlockSpec((MC2, D), lambda i: (i, 0)),
            pl.BlockSpec((MC2, D), lambda i: (i + H // MC2, 0)),
        ],
        out_specs=pl.BlockSpec((B, MC2), lambda i: (0, i)),
        out_shape=jax.ShapeDtypeStruct((B, H), jnp.int32),
    )(feature, memory, memory)


_mesh = plsc.VectorSubcoreMesh(core_axis_name="c", subcore_axis_name="s")


@functools.partial(
    pl.kernel,
    out_type=jax.ShapeDtypeStruct((B, K), jnp.float32),
    mesh=_mesh,
    compiler_params=pltpu.CompilerParams(needs_layout_passes=False),
    scratch_types=[
        pltpu.VMEM((H,), jnp.int32),
        pltpu.VMEM((H,), jnp.int32),
        pltpu.VMEM((K,), jnp.int32),
        pltpu.VMEM((K,), jnp.int32),
        pltpu.VMEM((ROWS_PER_TILE, K), jnp.float32),
        pltpu.SemaphoreType.DMA,
        pltpu.SemaphoreType.DMA,
        pltpu.SemaphoreType.DMA,
        pltpu.SemaphoreType.DMA,
    ],
)
def _gather(sims_hbm, idx_hbm, out_hbm, buf0, buf1, idx0, idx1, out_v,
            sem0, sem1, isem0, isem1):
    wid = lax.axis_index("s") * NC + lax.axis_index("c")
    b0 = wid * ROWS_PER_TILE
    bufs, sems = [buf0, buf1], [sem0, sem1]
    idxs, isems = [idx0, idx1], [isem0, isem1]

    pend_s = pltpu.async_copy(sims_hbm.at[b0], buf0, sem0)
    pend_i = pltpu.async_copy(idx_hbm.at[b0], idx0, isem0)
    for j in range(ROWS_PER_TILE):
        if j + 1 < ROWS_PER_TILE:
            nxt_s = pltpu.async_copy(
                sims_hbm.at[b0 + j + 1], bufs[(j + 1) % 2], sems[(j + 1) % 2]
            )
            nxt_i = pltpu.async_copy(
                idx_hbm.at[b0 + j + 1], idxs[(j + 1) % 2], isems[(j + 1) % 2]
            )
        pend_s.wait()
        pend_i.wait()
        buf, idr = bufs[j % 2], idxs[j % 2]

        @plsc.parallel_loop(0, K // 16, unroll=16)
        def _(t):
            iv = idr[pl.ds(t * 16, 16)]
            hi_half = lax.ge(iv, H)
            wi = jnp.where(hi_half, iv - H, iv)
            word = plsc.load_gather(buf, [wi])
            bits = jnp.where(hi_half, word, lax.shift_left(word, 16))
            f32bits = lax.bitwise_and(bits, jnp.int32(-65536))
            out_v[j, pl.ds(t * 16, 16)] = plsc.bitcast(f32bits, jnp.float32)
        if j + 1 < ROWS_PER_TILE:
            pend_s, pend_i = nxt_s, nxt_i
    pltpu.sync_copy(out_v, out_hbm.at[pl.ds(b0, ROWS_PER_TILE)])


def kernel(feature, memory, selected_neg_idx):
    idx = selected_neg_idx.reshape(B, K).astype(jnp.int32)
    sims_words = _sims_packed(feature, memory)
    out = _gather(sims_words, idx)
    return out[..., None]
